# baseline (speedup 1.0000x reference)
import functools

import jax
import jax.numpy as jnp
from jax import lax
from jax.experimental import pallas as pl
from jax.experimental.pallas import tpu as pltpu
from jax.experimental.pallas import tpu_sc as plsc

NC = 2
NS = 16
L = 16
NW = NC * NS

B = 16384
D = 32
NT = 1000000
BPW = B // NW
GPW = BPW // L

BN = 32768
RS = 196608
B0 = RS // BN
GTC = 25
NLO = (B0 + GTC) * BN
RPT = RS // NW
CSW = 768
NCH = RPT // CSW
CG = CSW // L

_mesh = plsc.VectorSubcoreMesh(core_axis_name="c", subcore_axis_name="s")


def _sweep_body(ut_ref, it_ref, wu_ref, wi_ref, su_ref, si_ref):
    su_ref[...] = jnp.dot(wu_ref[...], ut_ref[...],
                          preferred_element_type=jnp.float32)[0]
    si_ref[...] = jnp.dot(wi_ref[...], it_ref[...],
                          preferred_element_type=jnp.float32)[0]


_sweep = pl.pallas_call(
    _sweep_body,
    grid=(GTC,),
    in_specs=[
        pl.BlockSpec((D, BN), lambda i: (0, B0 + i)),
        pl.BlockSpec((D, BN), lambda i: (0, B0 + i)),
        pl.BlockSpec((1, D), lambda i: (0, 0)),
        pl.BlockSpec((1, D), lambda i: (0, 0)),
    ],
    out_specs=[
        pl.BlockSpec((BN,), lambda i: (B0 + i,)),
        pl.BlockSpec((BN,), lambda i: (B0 + i,)),
    ],
    out_shape=[jax.ShapeDtypeStruct((NLO,), jnp.float32)] * 2,
)


@functools.partial(
    pl.kernel,
    out_type=[jax.ShapeDtypeStruct((RS,), jnp.float32)] * 2,
    mesh=_mesh,
    scratch_types=[
        pltpu.VMEM((2, D, CSW), jnp.float32),
        pltpu.VMEM((2, D, CSW), jnp.float32),
        pltpu.VMEM((2 * D * L,), jnp.float32),
        pltpu.VMEM((RPT,), jnp.float32),
        pltpu.VMEM((RPT,), jnp.float32),
        pltpu.SemaphoreType.DMA,
        pltpu.SemaphoreType.DMA,
        pltpu.SemaphoreType.DMA,
        pltpu.SemaphoreType.DMA,
    ],
    compiler_params=pltpu.CompilerParams(needs_layout_passes=False),
)
def _sweep_sc(utab, itab, wb, su_hi, si_hi,
              ubuf, ibuf, w_v, su_v, si_v, su0, su1, si0, si1):
    wid = lax.axis_index("s") * NC + lax.axis_index("c")
    col0 = wid * RPT

    usems = (su0, su1)
    isems = (si0, si1)

    def fetch(c):
        s = c % 2
        off = col0 + c * CSW
        cu = pltpu.async_copy(
            utab.at[:, pl.ds(off, CSW)], ubuf.at[s], usems[s])
        ci = pltpu.async_copy(
            itab.at[:, pl.ds(off, CSW)], ibuf.at[s], isems[s])
        return cu, ci

    pend = fetch(0)
    pltpu.sync_copy(wb, w_v)

    for c in range(NCH):
        cu, ci = pend
        cu.wait()
        ci.wait()
        if c + 1 < NCH:
            pend = fetch(c + 1)
        ub = ubuf.at[c % 2]
        ib = ibuf.at[c % 2]

        def group(g, carry):
            sl = pl.ds(g * L, L)
            accu = ub[0, sl] * w_v[pl.ds(0, L)]
            acci = ib[0, sl] * w_v[pl.ds(D * L, L)]
            for d in range(1, D):
                accu = accu + ub[d, sl] * w_v[pl.ds(d * L, L)]
                acci = acci + ib[d, sl] * w_v[pl.ds((D + d) * L, L)]
            su_v[pl.ds(c * CSW + g * L, L)] = accu
            si_v[pl.ds(c * CSW + g * L, L)] = acci
            return carry

        lax.fori_loop(0, CG, group, 0)

    pltpu.sync_copy(su_v, su_hi.at[pl.ds(wid * RPT, RPT)])
    pltpu.sync_copy(si_v, si_hi.at[pl.ds(wid * RPT, RPT)])


@functools.partial(
    pl.kernel,
    out_type=jax.ShapeDtypeStruct((B,), jnp.float32),
    mesh=_mesh,
    scratch_types=[
        pltpu.VMEM((BPW,), jnp.int32),
        pltpu.VMEM((BPW,), jnp.int32),
        pltpu.VMEM((BPW,), jnp.int32),
        pltpu.VMEM((BPW,), jnp.int32),
        pltpu.VMEM((BPW,), jnp.float32),
        pltpu.VMEM((BPW,), jnp.float32),
        pltpu.VMEM((BPW,), jnp.float32),
        pltpu.VMEM((BPW,), jnp.float32),
        pltpu.VMEM((L,), jnp.float32),
        pltpu.VMEM((BPW,), jnp.float32),
        pltpu.SemaphoreType.DMA,
        pltpu.SemaphoreType.DMA,
    ],
    compiler_params=pltpu.CompilerParams(needs_layout_passes=False),
)
def _gather_sc(uids, iids, su_lo, si_lo, su_hi, si_hi, bb, out,
               uidx_v, iidx_v, uhi_v, ihi_v,
               gul_v, guh_v, gil_v, gih_v, b_v, out_v, sem_u, sem_i):
    wid = lax.axis_index("s") * NC + lax.axis_index("c")
    base = wid * BPW

    pltpu.sync_copy(uids.at[pl.ds(base, BPW)], uidx_v)
    pltpu.sync_copy(iids.at[pl.ds(base, BPW)], iidx_v)

    hi_max = jnp.full((L,), RS - 1, jnp.int32)
    split = jnp.full((L,), RS, jnp.int32)

    def build(g, carry):
        sl = pl.ds(g * L, L)
        uhi_v[sl] = jnp.minimum(uidx_v[sl], hi_max)
        ihi_v[sl] = jnp.minimum(iidx_v[sl], hi_max)
        return carry

    lax.fori_loop(0, GPW, build, 0)

    c1 = pltpu.async_copy(su_lo.at[uidx_v], gul_v, sem_u)
    c2 = pltpu.async_copy(su_hi.at[uhi_v], guh_v, sem_u)
    c3 = pltpu.async_copy(si_lo.at[iidx_v], gil_v, sem_i)
    c4 = pltpu.async_copy(si_hi.at[ihi_v], gih_v, sem_i)
    pltpu.sync_copy(bb, b_v)
    c1.wait()
    c2.wait()
    c3.wait()
    c4.wait()

    def group(g, carry):
        sl = pl.ds(g * L, L)
        su = jnp.where(uidx_v[sl] < split, guh_v[sl], gul_v[sl])
        si = jnp.where(iidx_v[sl] < split, gih_v[sl], gil_v[sl])
        out_v[sl] = su + si + b_v[...]
        return carry

    lax.fori_loop(0, GPW, group, 0)

    pltpu.sync_copy(out_v, out.at[pl.ds(base, BPW)])


def kernel(user_ids, item_ids, user_table, item_table, W, b):
    ut = user_table.T
    it = item_table.T
    wu = W[:, :D]
    wi = W[:, D:]
    wb = jnp.broadcast_to(W.reshape(2 * D, 1), (2 * D, L)).reshape(-1)
    su_lo, si_lo = _sweep(ut, it, wu, wi)
    su_hi, si_hi = _sweep_sc(ut, it, wb)
    bb = jnp.broadcast_to(b, (L,))
    out = _gather_sc(user_ids, item_ids, su_lo, si_lo, su_hi, si_hi, bb)
    return out.reshape(B, 1)


# speedup vs baseline: 1.5870x; 1.5870x over previous
import functools

import jax
import jax.numpy as jnp
from jax import lax
from jax.experimental import pallas as pl
from jax.experimental.pallas import tpu as pltpu
from jax.experimental.pallas import tpu_sc as plsc

NC = 2
NS = 16
L = 16
NW = NC * NS

B = 16384
D = 32
NT = 1000000
BPW = B // NW
GPW = BPW // L

BN = 32768
RS = 196608
B0 = RS // BN
GTC = 25
NLO = (B0 + GTC) * BN
RPT = RS // NW
CSW = 768
NCH = RPT // CSW
CG = CSW // L

_mesh = plsc.VectorSubcoreMesh(core_axis_name="c", subcore_axis_name="s")


def _sweep_body(ut_ref, it_ref, wu_ref, wi_ref, su_ref, si_ref):
    su_ref[...] = jnp.dot(wu_ref[...], ut_ref[...],
                          preferred_element_type=jnp.float32)[0]
    si_ref[...] = jnp.dot(wi_ref[...], it_ref[...],
                          preferred_element_type=jnp.float32)[0]


_sweep = pl.pallas_call(
    _sweep_body,
    grid=(GTC,),
    in_specs=[
        pl.BlockSpec((D, BN), lambda i: (0, B0 + i)),
        pl.BlockSpec((D, BN), lambda i: (0, B0 + i)),
        pl.BlockSpec((1, D), lambda i: (0, 0)),
        pl.BlockSpec((1, D), lambda i: (0, 0)),
    ],
    out_specs=[
        pl.BlockSpec((BN,), lambda i: (B0 + i,)),
        pl.BlockSpec((BN,), lambda i: (B0 + i,)),
    ],
    out_shape=[jax.ShapeDtypeStruct((NLO,), jnp.float32)] * 2,
)


@functools.partial(
    pl.kernel,
    out_type=[jax.ShapeDtypeStruct((NT,), jnp.float32)] * 2,
    mesh=_mesh,
    scratch_types=[
        pltpu.VMEM((2, D, CSW), jnp.float32),
        pltpu.VMEM((2, D, CSW), jnp.float32),
        pltpu.VMEM((2 * D * L,), jnp.float32),
        pltpu.VMEM((RPT,), jnp.float32),
        pltpu.VMEM((RPT,), jnp.float32),
        pltpu.SemaphoreType.DMA,
        pltpu.SemaphoreType.DMA,
        pltpu.SemaphoreType.DMA,
        pltpu.SemaphoreType.DMA,
    ],
    compiler_params=pltpu.CompilerParams(needs_layout_passes=False),
)
def _sweep_sc(utab, itab, wb, su_hi, si_hi,
              ubuf, ibuf, w_v, su_v, si_v, su0, su1, si0, si1):
    wid = lax.axis_index("s") * NC + lax.axis_index("c")
    col0 = wid * RPT

    usems = (su0, su1)
    isems = (si0, si1)

    def fetch(c):
        s = c % 2
        off = col0 + c * CSW
        cu = pltpu.async_copy(
            utab.at[:, pl.ds(off, CSW)], ubuf.at[s], usems[s])
        ci = pltpu.async_copy(
            itab.at[:, pl.ds(off, CSW)], ibuf.at[s], isems[s])
        return cu, ci

    pend = fetch(0)
    pltpu.sync_copy(wb, w_v)

    for c in range(NCH):
        cu, ci = pend
        cu.wait()
        ci.wait()
        if c + 1 < NCH:
            pend = fetch(c + 1)
        ub = ubuf.at[c % 2]
        ib = ibuf.at[c % 2]

        def group(g, carry):
            sl = pl.ds(g * L, L)
            accu = ub[0, sl] * w_v[pl.ds(0, L)]
            acci = ib[0, sl] * w_v[pl.ds(D * L, L)]
            for d in range(1, D):
                accu = accu + ub[d, sl] * w_v[pl.ds(d * L, L)]
                acci = acci + ib[d, sl] * w_v[pl.ds((D + d) * L, L)]
            su_v[pl.ds(c * CSW + g * L, L)] = accu
            si_v[pl.ds(c * CSW + g * L, L)] = acci
            return carry

        lax.fori_loop(0, CG, group, 0)

    pltpu.sync_copy(su_v, su_hi.at[pl.ds(wid * RPT, RPT)])
    pltpu.sync_copy(si_v, si_hi.at[pl.ds(wid * RPT, RPT)])


@functools.partial(
    pl.kernel,
    out_type=jax.ShapeDtypeStruct((B,), jnp.float32),
    mesh=_mesh,
    scratch_types=[
        pltpu.VMEM((BPW,), jnp.int32),
        pltpu.VMEM((BPW,), jnp.int32),
        pltpu.VMEM((BPW,), jnp.float32),
        pltpu.VMEM((BPW,), jnp.float32),
        pltpu.VMEM((BPW,), jnp.float32),
        pltpu.VMEM((BPW,), jnp.float32),
        pltpu.VMEM((L,), jnp.float32),
        pltpu.VMEM((BPW,), jnp.float32),
        pltpu.SemaphoreType.DMA,
        pltpu.SemaphoreType.DMA,
    ],
    compiler_params=pltpu.CompilerParams(needs_layout_passes=False),
)
def _gather_sc(uids, iids, su_lo, si_lo, su_hi, si_hi, bb, out,
               uidx_v, iidx_v,
               gul_v, guh_v, gil_v, gih_v, b_v, out_v, sem_u, sem_i):
    wid = lax.axis_index("s") * NC + lax.axis_index("c")
    base = wid * BPW

    pltpu.sync_copy(uids.at[pl.ds(base, BPW)], uidx_v)
    pltpu.sync_copy(iids.at[pl.ds(base, BPW)], iidx_v)

    split = jnp.full((L,), RS, jnp.int32)

    c1 = pltpu.async_copy(su_lo.at[uidx_v], gul_v, sem_u)
    c2 = pltpu.async_copy(su_hi.at[uidx_v], guh_v, sem_u)
    c3 = pltpu.async_copy(si_lo.at[iidx_v], gil_v, sem_i)
    c4 = pltpu.async_copy(si_hi.at[iidx_v], gih_v, sem_i)
    pltpu.sync_copy(bb, b_v)
    c1.wait()
    c2.wait()
    c3.wait()
    c4.wait()

    def group(g, carry):
        sl = pl.ds(g * L, L)
        su = jnp.where(uidx_v[sl] < split, guh_v[sl], gul_v[sl])
        si = jnp.where(iidx_v[sl] < split, gih_v[sl], gil_v[sl])
        out_v[sl] = su + si + b_v[...]
        return carry

    lax.fori_loop(0, GPW, group, 0)

    pltpu.sync_copy(out_v, out.at[pl.ds(base, BPW)])


def kernel(user_ids, item_ids, user_table, item_table, W, b):
    ut = user_table.T
    it = item_table.T
    wu = W[:, :D]
    wi = W[:, D:]
    wb = jnp.broadcast_to(W.reshape(2 * D, 1), (2 * D, L)).reshape(-1)
    su_hi, si_hi = _sweep_sc(ut, it, wb)
    su_lo, si_lo = _sweep(ut, it, wu, wi)
    bb = jnp.broadcast_to(b, (L,))
    out = _gather_sc(user_ids, item_ids, su_lo, si_lo, su_hi, si_hi, bb)
    return out.reshape(B, 1)


# speedup vs baseline: 1.6462x; 1.0373x over previous
import functools

import jax
import jax.numpy as jnp
from jax import lax
from jax.experimental import pallas as pl
from jax.experimental.pallas import tpu as pltpu
from jax.experimental.pallas import tpu_sc as plsc

NC = 2
NS = 16
L = 16
NW = NC * NS

B = 16384
D = 32
NT = 1000000
BPW = B // NW
GPW = BPW // L
BN = 35840

_mesh = plsc.VectorSubcoreMesh(core_axis_name="c", subcore_axis_name="s")


def _sweep_body(ut_ref, it_ref, wu_ref, wi_ref, su_ref, si_ref):
    su_ref[...] = jnp.dot(wu_ref[...], ut_ref[...],
                          preferred_element_type=jnp.float32)[0]
    si_ref[...] = jnp.dot(wi_ref[...], it_ref[...],
                          preferred_element_type=jnp.float32)[0]


_sweep = pl.pallas_call(
    _sweep_body,
    grid=(pl.cdiv(NT, BN),),
    in_specs=[
        pl.BlockSpec((D, BN), lambda i: (0, i)),
        pl.BlockSpec((D, BN), lambda i: (0, i)),
        pl.BlockSpec((1, D), lambda i: (0, 0)),
        pl.BlockSpec((1, D), lambda i: (0, 0)),
    ],
    out_specs=[
        pl.BlockSpec((BN,), lambda i: (i,)),
        pl.BlockSpec((BN,), lambda i: (i,)),
    ],
    out_shape=[jax.ShapeDtypeStruct((NT,), jnp.float32)] * 2,
)


@functools.partial(
    pl.kernel,
    out_type=jax.ShapeDtypeStruct((B,), jnp.float32),
    mesh=_mesh,
    scratch_types=[
        pltpu.VMEM((BPW,), jnp.int32),
        pltpu.VMEM((BPW,), jnp.int32),
        pltpu.VMEM((BPW,), jnp.float32),
        pltpu.VMEM((BPW,), jnp.float32),
        pltpu.VMEM((L,), jnp.float32),
        pltpu.VMEM((BPW,), jnp.float32),
        pltpu.SemaphoreType.DMA,
        pltpu.SemaphoreType.DMA,
    ],
    compiler_params=pltpu.CompilerParams(needs_layout_passes=False),
)
def _gather_sc(uids, iids, su, si, bb, out,
               uidx_v, iidx_v, sug_v, sig_v, b_v, out_v, sem_u, sem_i):
    wid = lax.axis_index("s") * NC + lax.axis_index("c")
    base = wid * BPW

    pltpu.sync_copy(uids.at[pl.ds(base, BPW)], uidx_v)
    pltpu.sync_copy(iids.at[pl.ds(base, BPW)], iidx_v)
    cu = pltpu.async_copy(su.at[uidx_v], sug_v, sem_u)
    ci = pltpu.async_copy(si.at[iidx_v], sig_v, sem_i)
    pltpu.sync_copy(bb, b_v)
    cu.wait()
    ci.wait()

    def group(g, carry):
        sl = pl.ds(g * L, L)
        out_v[sl] = sug_v[sl] + sig_v[sl] + b_v[...]
        return carry

    lax.fori_loop(0, GPW, group, 0)

    pltpu.sync_copy(out_v, out.at[pl.ds(base, BPW)])


def kernel(user_ids, item_ids, user_table, item_table, W, b):
    ut = user_table.T
    it = item_table.T
    wu = W[:, :D]
    wi = W[:, D:]
    su, si = _sweep(ut, it, wu, wi)
    bb = jnp.broadcast_to(b, (L,))
    out = _gather_sc(user_ids, item_ids, su, si, bb)
    return out.reshape(B, 1)
